# baseline (speedup 1.0000x reference)
import functools

import jax
import jax.numpy as jnp
from jax import lax
from jax.experimental import pallas as pl
from jax.experimental.pallas import tpu as pltpu
from jax.experimental.pallas import tpu_sc as plsc

VOCAB = 1000000
EMB = 64
HID = 128
OUT = 64
B = 16384
L = 50

NC = 2
NS = 16
NW = NC * NS
BAGS_PER_W = B // NW
IDX_PER_W = BAGS_PER_W * L
CHUNK = 2 * L
NCHUNK = IDX_PER_W // CHUNK

TW_BLK = 16384


def _fold_w1_body(t2_ref, w1s_ref, o_ref):
  o_ref[...] = jax.lax.dot_general(
      t2_ref[...], w1s_ref[...], (((0,), (0,)), ((), ())),
      preferred_element_type=jnp.float32,
  )


def _fold_w1(table_t, w1s):
  grid = (pl.cdiv(VOCAB, TW_BLK),)
  return pl.pallas_call(
      _fold_w1_body,
      grid=grid,
      compiler_params=pltpu.CompilerParams(fuse_transposed_lhs_in_matmul=True),
      in_specs=[
          pl.BlockSpec((EMB, TW_BLK), lambda i: (0, i)),
          pl.BlockSpec((EMB, HID), lambda i: (0, 0)),
      ],
      out_specs=pl.BlockSpec((TW_BLK, HID), lambda i: (i, 0)),
      out_shape=jax.ShapeDtypeStruct((VOCAB, HID), jnp.float32),
  )(table_t, w1s)


def _emb_bag_sum(idx_grouped, tw):
  mesh = plsc.VectorSubcoreMesh(core_axis_name="c", subcore_axis_name="s")

  @functools.partial(
      pl.kernel,
      mesh=mesh,
      compiler_params=pltpu.CompilerParams(use_tc_tiling_on_sc=True),
      out_type=jax.ShapeDtypeStruct((B, HID), jnp.float32),
      scratch_types=[
          pltpu.VMEM((NCHUNK, CHUNK), jnp.int32),
          pltpu.VMEM((CHUNK, HID), jnp.float32),
          pltpu.VMEM((CHUNK, HID), jnp.float32),
          pltpu.VMEM((CHUNK, HID), jnp.float32),
          pltpu.VMEM((CHUNK, HID), jnp.float32),
          pltpu.VMEM((64, HID), jnp.float32),
          pltpu.SemaphoreType.DMA,
          pltpu.SemaphoreType.DMA,
      ],
  )
  def k(idx_hbm, tw_hbm, out_hbm, idx_v, rows0_v, rows1_v, rows2_v, rows3_v,
        out_v, sem0, sem1):
    cid = lax.axis_index("c")
    sid = lax.axis_index("s")
    wid = sid * NC + cid
    bufs = (rows0_v, rows1_v, rows2_v, rows3_v)
    sems = (sem0, sem0, sem1, sem1)

    pltpu.sync_copy(idx_hbm.at[wid], idx_v)

    def gather(c, buf, sem):
      return pltpu.make_async_copy(tw_hbm.at[idx_v.at[c]], buf, sem)

    def process(c, buf):
      for h in range(2):

        def row_body(j, accs):
          r = h * L + j
          return tuple(
              accs[q] + buf[r, pl.ds(q * 16, 16)] for q in range(8)
          )

        zero8 = tuple(jnp.zeros((16,), jnp.float32) for _ in range(8))
        accs = lax.fori_loop(0, L, row_body, zero8, unroll=5)
        slot = lax.rem(2 * c + h, 64)
        for q in range(8):
          out_v[slot, pl.ds(q * 16, 16)] = accs[q]

    for b in range(4):
      gather(b, bufs[b], sems[b]).start()

    def outer(s, _):
      def inner(gg, _):
        g = 8 * s + gg
        base = 4 * g
        for pair in range(2):
          for b in (2 * pair, 2 * pair + 1):
            gather(base + b, bufs[b], sems[b]).wait()
          for b in (2 * pair, 2 * pair + 1):
            process(base + b, bufs[b])

          @pl.when(g < NCHUNK // 4 - 1)
          def _():
            for b in (2 * pair, 2 * pair + 1):
              gather(base + b + 4, bufs[b], sems[b]).start()

        return 0

      lax.fori_loop(0, 8, inner, 0)
      pltpu.sync_copy(
          out_v, out_hbm.at[pl.ds(wid * BAGS_PER_W + s * 64, 64)]
      )
      return 0

    lax.fori_loop(0, 8, outer, 0)

  return k(idx_grouped, tw)


def _mlp_body(x_ref, b1_ref, w2_ref, b2_ref, w5_ref, b5_ref, o_ref):
  h = jnp.maximum(x_ref[...] + b1_ref[...], 0.0)
  h = jnp.dot(h, w2_ref[...], preferred_element_type=jnp.float32) + b2_ref[...]
  h = jnp.maximum(h, 0.0)
  o_ref[...] = (
      jnp.dot(h, w5_ref[...], preferred_element_type=jnp.float32) + b5_ref[...]
  )


def _mlp(x, b1, w2t, b2, w5t, b5):
  BLK = 2048
  grid = (B // BLK,)
  return pl.pallas_call(
      _mlp_body,
      grid=grid,
      in_specs=[
          pl.BlockSpec((BLK, HID), lambda i: (i, 0)),
          pl.BlockSpec((1, HID), lambda i: (0, 0)),
          pl.BlockSpec((HID, HID), lambda i: (0, 0)),
          pl.BlockSpec((1, HID), lambda i: (0, 0)),
          pl.BlockSpec((HID, OUT), lambda i: (0, 0)),
          pl.BlockSpec((1, OUT), lambda i: (0, 0)),
      ],
      out_specs=pl.BlockSpec((BLK, OUT), lambda i: (i, 0)),
      out_shape=jax.ShapeDtypeStruct((B, OUT), jnp.float32),
  )(x, b1.reshape(1, HID), w2t, b2.reshape(1, HID), w5t, b5.reshape(1, OUT))


def kernel(text, emb_table, w1, b1, w2, b2, w5, b5):
  table_t = emb_table.T
  tw = _fold_w1(table_t, w1.T / L)
  idx = text.reshape(NW, NCHUNK, CHUNK)
  bag_sums = _emb_bag_sum(idx, tw)
  return _mlp(bag_sums, b1, w2.T, b2, w5.T, b5)


# speedup vs baseline: 1.0125x; 1.0125x over previous
import functools

import jax
import jax.numpy as jnp
from jax import lax
from jax.experimental import pallas as pl
from jax.experimental.pallas import tpu as pltpu
from jax.experimental.pallas import tpu_sc as plsc

VOCAB = 1000000
EMB = 64
HID = 128
OUT = 64
B = 16384
L = 50

NC = 2
NS = 16
NW = NC * NS
BAGS_PER_W = B // NW
IDX_PER_W = BAGS_PER_W * L
CHUNK = 2 * L
NCHUNK = IDX_PER_W // CHUNK

TW_BLK = 32768


def _fold_w1_body(t2_ref, w1s_ref, o_ref):
  o_ref[...] = jax.lax.dot_general(
      t2_ref[...], w1s_ref[...], (((0,), (0,)), ((), ())),
      preferred_element_type=jnp.float32,
  )


def _fold_w1(table_t, w1s):
  grid = (pl.cdiv(VOCAB, TW_BLK),)
  return pl.pallas_call(
      _fold_w1_body,
      grid=grid,
      compiler_params=pltpu.CompilerParams(fuse_transposed_lhs_in_matmul=True),
      in_specs=[
          pl.BlockSpec((EMB, TW_BLK), lambda i: (0, i)),
          pl.BlockSpec((EMB, HID), lambda i: (0, 0)),
      ],
      out_specs=pl.BlockSpec((TW_BLK, HID), lambda i: (i, 0)),
      out_shape=jax.ShapeDtypeStruct((VOCAB, HID), jnp.float32),
  )(table_t, w1s)


def _emb_bag_sum(idx_grouped, tw):
  mesh = plsc.VectorSubcoreMesh(core_axis_name="c", subcore_axis_name="s")

  @functools.partial(
      pl.kernel,
      mesh=mesh,
      compiler_params=pltpu.CompilerParams(use_tc_tiling_on_sc=True),
      out_type=jax.ShapeDtypeStruct((B, HID), jnp.float32),
      scratch_types=[
          pltpu.VMEM((NCHUNK, CHUNK), jnp.int32),
          pltpu.VMEM((CHUNK, HID), jnp.float32),
          pltpu.VMEM((CHUNK, HID), jnp.float32),
          pltpu.VMEM((CHUNK, HID), jnp.float32),
          pltpu.VMEM((CHUNK, HID), jnp.float32),
          pltpu.VMEM((64, HID), jnp.float32),
          pltpu.SemaphoreType.DMA,
          pltpu.SemaphoreType.DMA,
      ],
  )
  def k(idx_hbm, tw_hbm, out_hbm, idx_v, rows0_v, rows1_v, rows2_v, rows3_v,
        out_v, sem0, sem1):
    cid = lax.axis_index("c")
    sid = lax.axis_index("s")
    wid = sid * NC + cid
    bufs = (rows0_v, rows1_v, rows2_v, rows3_v)
    sems = (sem0, sem0, sem1, sem1)

    pltpu.sync_copy(idx_hbm.at[wid], idx_v)

    def gather(c, buf, sem):
      return pltpu.make_async_copy(tw_hbm.at[idx_v.at[c]], buf, sem)

    def process(c, buf):
      for h in range(2):

        def row_body(j, accs):
          r = h * L + j
          return tuple(
              accs[q] + buf[r, pl.ds(q * 16, 16)] for q in range(8)
          )

        zero8 = tuple(jnp.zeros((16,), jnp.float32) for _ in range(8))
        accs = lax.fori_loop(0, L, row_body, zero8, unroll=5)
        slot = lax.rem(2 * c + h, 64)
        for q in range(8):
          out_v[slot, pl.ds(q * 16, 16)] = accs[q]

    for b in range(4):
      gather(b, bufs[b], sems[b]).start()

    def outer(s, _):
      def inner(gg, _):
        g = 8 * s + gg
        base = 4 * g
        for pair in range(2):
          for b in (2 * pair, 2 * pair + 1):
            gather(base + b, bufs[b], sems[b]).wait()
          for b in (2 * pair, 2 * pair + 1):
            process(base + b, bufs[b])

          @pl.when(g < NCHUNK // 4 - 1)
          def _():
            for b in (2 * pair, 2 * pair + 1):
              gather(base + b + 4, bufs[b], sems[b]).start()

        return 0

      lax.fori_loop(0, 8, inner, 0)
      pltpu.sync_copy(
          out_v, out_hbm.at[pl.ds(wid * BAGS_PER_W + s * 64, 64)]
      )
      return 0

    lax.fori_loop(0, 8, outer, 0)

  return k(idx_grouped, tw)


def _mlp_body(x_ref, b1_ref, w2_ref, b2_ref, w5_ref, b5_ref, o_ref):
  h = jnp.maximum(x_ref[...] + b1_ref[...], 0.0)
  h = jnp.dot(h, w2_ref[...], preferred_element_type=jnp.float32) + b2_ref[...]
  h = jnp.maximum(h, 0.0)
  o_ref[...] = (
      jnp.dot(h, w5_ref[...], preferred_element_type=jnp.float32) + b5_ref[...]
  )


def _mlp(x, b1, w2t, b2, w5t, b5):
  BLK = 2048
  grid = (B // BLK,)
  return pl.pallas_call(
      _mlp_body,
      grid=grid,
      in_specs=[
          pl.BlockSpec((BLK, HID), lambda i: (i, 0)),
          pl.BlockSpec((1, HID), lambda i: (0, 0)),
          pl.BlockSpec((HID, HID), lambda i: (0, 0)),
          pl.BlockSpec((1, HID), lambda i: (0, 0)),
          pl.BlockSpec((HID, OUT), lambda i: (0, 0)),
          pl.BlockSpec((1, OUT), lambda i: (0, 0)),
      ],
      out_specs=pl.BlockSpec((BLK, OUT), lambda i: (i, 0)),
      out_shape=jax.ShapeDtypeStruct((B, OUT), jnp.float32),
  )(x, b1.reshape(1, HID), w2t, b2.reshape(1, HID), w5t, b5.reshape(1, OUT))


def kernel(text, emb_table, w1, b1, w2, b2, w5, b5):
  table_t = emb_table.T
  tw = _fold_w1(table_t, w1.T / L)
  idx = text.reshape(NW, NCHUNK, CHUNK)
  bag_sums = _emb_bag_sum(idx, tw)
  return _mlp(bag_sums, b1, w2.T, b2, w5.T, b5)
